# Initial kernel scaffold; baseline (speedup 1.0000x reference)
import jax
import jax.numpy as jnp
from jax.experimental import pallas as pl


def kernel(token_ids, embedding_table):
    raise NotImplementedError("write your pallas kernel here")


# speedup vs baseline: 4.9496x; 4.9496x over previous
import jax
import jax.numpy as jnp
from jax import lax
from jax.experimental import pallas as pl
from jax.experimental.pallas import tpu as pltpu
from jax.experimental.pallas import tpu_sc as plsc

_info = plsc.get_sparse_core_info()
_NC, _NS = _info.num_cores, _info.num_subcores
_NW = _NC * _NS

_CHUNK = 2048


def _gather_body(nchunk):
    def body(table_hbm, idx_hbm, out_hbm, idx_v, rows_v, sem):
        wid = lax.axis_index("s") * _NC + lax.axis_index("c")
        wbase = wid * (nchunk * _CHUNK)

        def step(g, carry):
            base = wbase + g * _CHUNK
            pltpu.sync_copy(idx_hbm.at[pl.ds(base, _CHUNK)], idx_v)
            pltpu.async_copy(table_hbm.at[idx_v], rows_v, sem).wait()
            pltpu.sync_copy(rows_v, out_hbm.at[pl.ds(base, _CHUNK)])
            return carry

        lax.fori_loop(0, nchunk, step, 0)

    return body


def kernel(token_ids, embedding_table):
    batch, hist = token_ids.shape
    n_emb, d = embedding_table.shape
    b = batch * hist
    assert b % (_NW * _CHUNK) == 0
    nchunk = b // (_NW * _CHUNK)

    flat_ids = token_ids.reshape(b).astype(jnp.int32)
    mesh = plsc.VectorSubcoreMesh(core_axis_name="c", subcore_axis_name="s")
    out = pl.kernel(
        _gather_body(nchunk),
        out_type=jax.ShapeDtypeStruct((b, d), jnp.float32),
        mesh=mesh,
        compiler_params=pltpu.CompilerParams(use_tc_tiling_on_sc=False),
        scratch_types=[
            pltpu.VMEM((_CHUNK,), jnp.int32),
            pltpu.VMEM((_CHUNK, d), jnp.float32),
            pltpu.SemaphoreType.DMA,
        ],
    )(embedding_table, flat_ids)
    return out.reshape(batch, hist, d)


# speedup vs baseline: 5.0400x; 1.0183x over previous
import jax
import jax.numpy as jnp
from jax import lax
from jax.experimental import pallas as pl
from jax.experimental.pallas import tpu as pltpu
from jax.experimental.pallas import tpu_sc as plsc

_info = plsc.get_sparse_core_info()
_NC, _NS = _info.num_cores, _info.num_subcores
_NW = _NC * _NS

_CHUNK = 1600


def _gather_body(nchunk):
    c = _CHUNK

    def body(table, idx_hbm, out, ia, ib, ra, rb, isa, isb, gsa, gsb, wsa, wsb):
        wid = lax.axis_index("s") * _NC + lax.axis_index("c")
        wbase = wid * (nchunk * c)

        buf_a = (ia, ra, isa, gsa, wsa)
        buf_b = (ib, rb, isb, gsb, wsb)

        def off(g):
            return wbase + g * c

        def step(g, cur, nxt, wait_prev_wb, gather_next, idx_next):
            i_c, r_c, is_c, gs_c, ws_c = cur
            i_n, r_n, is_n, gs_n, ws_n = nxt
            pltpu.make_async_copy(table.at[i_c], r_c, gs_c).wait()
            pltpu.async_copy(r_c, out.at[pl.ds(off(g), c)], ws_c)
            if wait_prev_wb:
                pltpu.make_async_copy(r_n, out.at[pl.ds(wbase, c)], ws_n).wait()
            if gather_next:
                pltpu.make_async_copy(idx_hbm.at[pl.ds(wbase, c)], i_n, is_n).wait()
                pltpu.async_copy(table.at[i_n], r_n, gs_n)
            if idx_next:
                pltpu.async_copy(idx_hbm.at[pl.ds(off(g + 2), c)], i_c, is_c)

        pltpu.async_copy(idx_hbm.at[pl.ds(off(0), c)], ia, isa)
        pltpu.make_async_copy(idx_hbm.at[pl.ds(wbase, c)], ia, isa).wait()
        pltpu.async_copy(table.at[ia], ra, gsa)
        pltpu.async_copy(idx_hbm.at[pl.ds(off(1), c)], ib, isb)

        step(0, buf_a, buf_b, False, True, True)
        step(1, buf_b, buf_a, True, True, True)

        def pair(i, carry):
            g0 = 2 * i
            step(g0, buf_a, buf_b, True, True, True)
            step(g0 + 1, buf_b, buf_a, True, True, True)
            return carry

        lax.fori_loop(1, nchunk // 2 - 1, pair, 0)

        step(nchunk - 2, buf_a, buf_b, True, True, False)
        step(nchunk - 1, buf_b, buf_a, True, False, False)

        pltpu.make_async_copy(rb, out.at[pl.ds(wbase, c)], wsb).wait()

    return body


def kernel(token_ids, embedding_table):
    batch, hist = token_ids.shape
    n_emb, d = embedding_table.shape
    b = batch * hist
    assert b % (_NW * _CHUNK) == 0
    nchunk = b // (_NW * _CHUNK)
    assert nchunk % 2 == 0 and nchunk >= 6

    flat_ids = token_ids.reshape(b).astype(jnp.int32)
    mesh = plsc.VectorSubcoreMesh(core_axis_name="c", subcore_axis_name="s")
    out = pl.kernel(
        _gather_body(nchunk),
        out_type=jax.ShapeDtypeStruct((b, d), jnp.float32),
        mesh=mesh,
        compiler_params=pltpu.CompilerParams(use_tc_tiling_on_sc=False),
        scratch_types=[
            pltpu.VMEM((_CHUNK,), jnp.int32),
            pltpu.VMEM((_CHUNK,), jnp.int32),
            pltpu.VMEM((_CHUNK, d), jnp.float32),
            pltpu.VMEM((_CHUNK, d), jnp.float32),
            pltpu.SemaphoreType.DMA,
            pltpu.SemaphoreType.DMA,
            pltpu.SemaphoreType.DMA,
            pltpu.SemaphoreType.DMA,
            pltpu.SemaphoreType.DMA,
            pltpu.SemaphoreType.DMA,
        ],
    )(embedding_table, flat_ids)
    return out.reshape(batch, hist, d)


# speedup vs baseline: 5.0520x; 1.0024x over previous
import jax
import jax.numpy as jnp
from jax import lax
from jax.experimental import pallas as pl
from jax.experimental.pallas import tpu as pltpu
from jax.experimental.pallas import tpu_sc as plsc

_info = plsc.get_sparse_core_info()
_NC, _NS = _info.num_cores, _info.num_subcores
_NW = _NC * _NS

_NBUF = 4
_CHUNK = 800


def _gather_body(nchunk):
    c, nb = _CHUNK, _NBUF
    k = nb - 1

    def body(table, idx_hbm, out, *scratch):
        ibufs = scratch[0:nb]
        rbufs = scratch[nb:2 * nb]
        isems = scratch[2 * nb:3 * nb]
        gsems = scratch[3 * nb:4 * nb]
        wsems = scratch[4 * nb:5 * nb]

        wid = lax.axis_index("s") * _NC + lax.axis_index("c")
        wbase = wid * (nchunk * c)

        def off(g):
            return wbase + g * c

        def idx_start(g, j):
            pltpu.async_copy(idx_hbm.at[pl.ds(off(g), c)], ibufs[j], isems[j])

        def idx_wait(j):
            pltpu.make_async_copy(idx_hbm.at[pl.ds(wbase, c)], ibufs[j], isems[j]).wait()

        def gat_start(j):
            pltpu.async_copy(table.at[ibufs[j]], rbufs[j], gsems[j])

        def gat_wait(j):
            pltpu.make_async_copy(table.at[ibufs[j]], rbufs[j], gsems[j]).wait()

        def wb_start(g, j):
            pltpu.async_copy(rbufs[j], out.at[pl.ds(off(g), c)], wsems[j])

        def wb_wait(j):
            pltpu.make_async_copy(rbufs[j], out.at[pl.ds(wbase, c)], wsems[j]).wait()

        def step(g, j, wait_prev, start_g, start_i):
            gat_wait(j)
            wb_start(g, j)
            if wait_prev:
                wb_wait((j - 1) % nb)
            if start_g:
                jj = (j + k) % nb
                idx_wait(jj)
                gat_start(jj)
            if start_i:
                idx_start(g + nb, j)

        for g in range(nb):
            idx_start(g, g)
        for g in range(k):
            idx_wait(g)
            gat_start(g)

        for j in range(nb):
            step(j, j, j >= 1, j + k < nchunk, j + nb < nchunk)

        nblocks = nchunk // nb

        def block(i, carry):
            g0 = i * nb
            for j in range(nb):
                step(g0 + j, j, True, True, True)
            return carry

        lax.fori_loop(1, nblocks - 1, block, 0)

        for j in range(nb):
            g = nchunk - nb + j
            step(g, j, True, g + k < nchunk, g + nb < nchunk)

        wb_wait(nb - 1)

    return body


def kernel(token_ids, embedding_table):
    batch, hist = token_ids.shape
    n_emb, d = embedding_table.shape
    b = batch * hist
    assert b % (_NW * _CHUNK) == 0
    nchunk = b // (_NW * _CHUNK)
    assert nchunk % _NBUF == 0 and nchunk // _NBUF >= 3

    flat_ids = token_ids.reshape(b).astype(jnp.int32)
    mesh = plsc.VectorSubcoreMesh(core_axis_name="c", subcore_axis_name="s")
    scratch = (
        [pltpu.VMEM((_CHUNK,), jnp.int32) for _ in range(_NBUF)]
        + [pltpu.VMEM((_CHUNK, d), jnp.float32) for _ in range(_NBUF)]
        + [pltpu.SemaphoreType.DMA for _ in range(3 * _NBUF)]
    )
    out = pl.kernel(
        _gather_body(nchunk),
        out_type=jax.ShapeDtypeStruct((b, d), jnp.float32),
        mesh=mesh,
        compiler_params=pltpu.CompilerParams(use_tc_tiling_on_sc=False),
        scratch_types=scratch,
    )(embedding_table, flat_ids)
    return out.reshape(batch, hist, d)
